# baseline (device time: 269927 ns/iter reference)
import jax
import jax.numpy as jnp
from jax import lax
from jax.experimental import pallas as pl
from jax.experimental.pallas import tpu as pltpu

N_DEV = 4
M = 4096
N = 2048
KS = 1024
MQ = M // N_DEV
RC = 4
RCM = MQ // RC


def _body(xl_ref, w_ref, ybuf_ref, scale_ref,
          recv_ref, sbuf_ref, yown_ref, amax_ref,
          rs_send, rs_recv, ax_send, ax_recv, ag_send, ag_recv):
    my = lax.axis_index("i")
    right = lax.rem(my + 1, N_DEV)
    left = lax.rem(my + N_DEV - 1, N_DEV)

    barrier = pltpu.get_barrier_semaphore()
    for nbr in (left, right):
        pl.semaphore_signal(barrier, inc=1, device_id=(nbr,),
                            device_id_type=pl.DeviceIdType.MESH)
    pl.semaphore_wait(barrier, 2)

    for r in range(RC):
        a, b = r * RCM, (r + 1) * RCM
        sbuf_ref[a:b, :] = jnp.dot(
            xl_ref[0, a:b, :], w_ref[:, :],
            preferred_element_type=jnp.float32).astype(jnp.bfloat16)

    for s in range(N_DEV - 1):
        src = sbuf_ref if s == 0 else recv_ref.at[s - 1]
        rdma = pltpu.make_async_remote_copy(
            src_ref=src,
            dst_ref=recv_ref.at[s],
            send_sem=rs_send.at[s],
            recv_sem=rs_recv.at[s],
            device_id=(right,),
            device_id_type=pl.DeviceIdType.MESH,
        )
        rdma.start()
        for r in range(RC):
            a, b = r * RCM, (r + 1) * RCM
            yown_ref[a:b, :] = jnp.dot(
                xl_ref[s + 1, a:b, :], w_ref[:, :],
                preferred_element_type=jnp.float32)
        rdma.wait()
        for r in range(RC):
            a, b = r * RCM, (r + 1) * RCM
            tot = recv_ref[s, a:b, :].astype(jnp.float32) + yown_ref[a:b, :]
            if s < N_DEV - 2:
                recv_ref[s, a:b, :] = tot.astype(jnp.bfloat16)
            else:
                yown_ref[a:b, :] = tot

    am = jnp.maximum(jnp.max(yown_ref[:, :]), 0.0)
    amax_ref[0, :, :] = jnp.full((8, 128), am, dtype=jnp.float32)
    for t in range(N_DEV - 1):
        rdma = pltpu.make_async_remote_copy(
            src_ref=amax_ref.at[t],
            dst_ref=amax_ref.at[t + 1],
            send_sem=ax_send.at[t],
            recv_sem=ax_recv.at[t],
            device_id=(right,),
            device_id_type=pl.DeviceIdType.MESH,
        )
        rdma.start()
        rdma.wait()
    g_amax = jnp.max(amax_ref[:, :, :])
    scale = g_amax / 127.0
    scale_ref[0, 0] = scale

    for r in range(RC):
        a, b = r * RCM, (r + 1) * RCM
        y = jnp.maximum(yown_ref[a:b, :], 0.0)
        q = jnp.clip(jnp.round(y / scale), 0.0, 127.0)
        ybuf_ref[0, a:b, :] = q.astype(jnp.int8)
    for t in range(N_DEV - 1):
        rdma = pltpu.make_async_remote_copy(
            src_ref=ybuf_ref.at[t],
            dst_ref=ybuf_ref.at[t + 1],
            send_sem=ag_send.at[t],
            recv_sem=ag_recv.at[t],
            device_id=(right,),
            device_id_type=pl.DeviceIdType.MESH,
        )
        rdma.start()
        rdma.wait()


def kernel(x, w_mat):
    x16 = x.astype(jnp.bfloat16)
    w16 = w_mat.astype(jnp.bfloat16)
    my = lax.axis_index("i")

    sidx = (my - jnp.arange(N_DEV)) % N_DEV
    xl = x16.reshape(N_DEV, MQ, KS)[sidx]

    ybuf, scale = pl.pallas_call(
        _body,
        out_shape=[
            jax.ShapeDtypeStruct((N_DEV, MQ, N), jnp.int8),
            jax.ShapeDtypeStruct((1, 1), jnp.float32),
        ],
        in_specs=[
            pl.BlockSpec(memory_space=pltpu.VMEM),
            pl.BlockSpec(memory_space=pltpu.VMEM),
        ],
        out_specs=[
            pl.BlockSpec(memory_space=pltpu.VMEM),
            pl.BlockSpec(memory_space=pltpu.SMEM),
        ],
        scratch_shapes=[
            pltpu.VMEM((N_DEV - 1, MQ, N), jnp.bfloat16),
            pltpu.VMEM((MQ, N), jnp.bfloat16),
            pltpu.VMEM((MQ, N), jnp.float32),
            pltpu.VMEM((N_DEV, 8, 128), jnp.float32),
            pltpu.SemaphoreType.DMA((N_DEV - 1,)),
            pltpu.SemaphoreType.DMA((N_DEV - 1,)),
            pltpu.SemaphoreType.DMA((N_DEV - 1,)),
            pltpu.SemaphoreType.DMA((N_DEV - 1,)),
            pltpu.SemaphoreType.DMA((N_DEV - 1,)),
            pltpu.SemaphoreType.DMA((N_DEV - 1,)),
        ],
        compiler_params=pltpu.CompilerParams(collective_id=0),
    )(xl, w16)

    oidx = (my + 1 - jnp.arange(N_DEV)) % N_DEV
    out = ybuf[oidx].reshape(M, N).astype(jnp.float32) * scale[0, 0]
    return out


# device time: 167512 ns/iter; 1.6114x vs baseline; 1.6114x over previous
import jax
import jax.numpy as jnp
from jax import lax
from jax.experimental import pallas as pl
from jax.experimental.pallas import tpu as pltpu

N_DEV = 4
M = 4096
N = 2048
NH = N // 2
KS = 1024
MQ = M // N_DEV
RC = 4
RCM = MQ // RC


def _body(xl_ref, w_ref, ycw_ref, yccw_ref, scale_ref,
          rcv_cw, rcv_ccw, sb_cw, sb_ccw, own_cw, own_ccw, amax_ref,
          rs_s_cw, rs_r_cw, rs_s_ccw, rs_r_ccw,
          ax_s, ax_r, ag_s_cw, ag_r_cw, ag_s_ccw, ag_r_ccw):
    my = lax.axis_index("i")
    right = lax.rem(my + 1, N_DEV)
    left = lax.rem(my + N_DEV - 1, N_DEV)

    barrier = pltpu.get_barrier_semaphore()
    for nbr in (left, right):
        pl.semaphore_signal(barrier, inc=1, device_id=(nbr,),
                            device_id_type=pl.DeviceIdType.MESH)
    pl.semaphore_wait(barrier, 2)

    def qdot(block, colhalf, r):
        a, b = r * RCM, (r + 1) * RCM
        return jnp.dot(xl_ref[block, a:b, :],
                       w_ref[:, colhalf * NH:(colhalf + 1) * NH],
                       preferred_element_type=jnp.float32)

    for r in range(RC):
        a, b = r * RCM, (r + 1) * RCM
        sb_cw[a:b, :] = qdot(0, 0, r).astype(jnp.bfloat16)
        sb_ccw[a:b, :] = qdot(0, 1, r).astype(jnp.bfloat16)

    for s in range(N_DEV - 1):
        cw = pltpu.make_async_remote_copy(
            src_ref=sb_cw if s == 0 else rcv_cw.at[s - 1],
            dst_ref=rcv_cw.at[s],
            send_sem=rs_s_cw.at[s], recv_sem=rs_r_cw.at[s],
            device_id=(right,), device_id_type=pl.DeviceIdType.MESH)
        ccw = pltpu.make_async_remote_copy(
            src_ref=sb_ccw if s == 0 else rcv_ccw.at[s - 1],
            dst_ref=rcv_ccw.at[s],
            send_sem=rs_s_ccw.at[s], recv_sem=rs_r_ccw.at[s],
            device_id=(left,), device_id_type=pl.DeviceIdType.MESH)
        cw.start()
        ccw.start()
        for r in range(RC):
            a, b = r * RCM, (r + 1) * RCM
            own_cw[a:b, :] = qdot(s + 1, 0, r)
            own_ccw[a:b, :] = qdot(3 - s, 1, r)
        cw.wait()
        ccw.wait()
        for r in range(RC):
            a, b = r * RCM, (r + 1) * RCM
            tcw = rcv_cw[s, a:b, :].astype(jnp.float32) + own_cw[a:b, :]
            tccw = rcv_ccw[s, a:b, :].astype(jnp.float32) + own_ccw[a:b, :]
            if s < N_DEV - 2:
                rcv_cw[s, a:b, :] = tcw.astype(jnp.bfloat16)
                rcv_ccw[s, a:b, :] = tccw.astype(jnp.bfloat16)
            else:
                own_cw[a:b, :] = tcw
                own_ccw[a:b, :] = tccw

    am = jnp.maximum(jnp.maximum(jnp.max(own_cw[:, :]),
                                 jnp.max(own_ccw[:, :])), 0.0)
    amax_ref[0, :, :] = jnp.full((8, 128), am, dtype=jnp.float32)
    for t in range(N_DEV - 1):
        rdma = pltpu.make_async_remote_copy(
            src_ref=amax_ref.at[t], dst_ref=amax_ref.at[t + 1],
            send_sem=ax_s.at[t], recv_sem=ax_r.at[t],
            device_id=(right,), device_id_type=pl.DeviceIdType.MESH)
        rdma.start()
        rdma.wait()
    g_amax = jnp.max(amax_ref[:, :, :])
    scale = g_amax / 127.0
    scale_ref[0, 0] = scale

    for r in range(RC):
        a, b = r * RCM, (r + 1) * RCM
        qcw = jnp.clip(jnp.round(jnp.maximum(own_cw[a:b, :], 0.0) / scale),
                       0.0, 127.0)
        qccw = jnp.clip(jnp.round(jnp.maximum(own_ccw[a:b, :], 0.0) / scale),
                        0.0, 127.0)
        ycw_ref[0, a:b, :] = qcw.astype(jnp.int8)
        yccw_ref[0, a:b, :] = qccw.astype(jnp.int8)
    for t in range(N_DEV - 1):
        cw = pltpu.make_async_remote_copy(
            src_ref=ycw_ref.at[t], dst_ref=ycw_ref.at[t + 1],
            send_sem=ag_s_cw.at[t], recv_sem=ag_r_cw.at[t],
            device_id=(right,), device_id_type=pl.DeviceIdType.MESH)
        ccw = pltpu.make_async_remote_copy(
            src_ref=yccw_ref.at[t], dst_ref=yccw_ref.at[t + 1],
            send_sem=ag_s_ccw.at[t], recv_sem=ag_r_ccw.at[t],
            device_id=(left,), device_id_type=pl.DeviceIdType.MESH)
        cw.start()
        ccw.start()
        cw.wait()
        ccw.wait()


def kernel(x, w_mat):
    x16 = x.astype(jnp.bfloat16)
    w16 = w_mat.astype(jnp.bfloat16)
    my = lax.axis_index("i")

    sidx = (my - jnp.arange(N_DEV)) % N_DEV
    xl = x16.reshape(N_DEV, MQ, KS)[sidx]

    ycw, yccw, scale = pl.pallas_call(
        _body,
        out_shape=[
            jax.ShapeDtypeStruct((N_DEV, MQ, NH), jnp.int8),
            jax.ShapeDtypeStruct((N_DEV, MQ, NH), jnp.int8),
            jax.ShapeDtypeStruct((1, 1), jnp.float32),
        ],
        in_specs=[
            pl.BlockSpec(memory_space=pltpu.VMEM),
            pl.BlockSpec(memory_space=pltpu.VMEM),
        ],
        out_specs=[
            pl.BlockSpec(memory_space=pltpu.VMEM),
            pl.BlockSpec(memory_space=pltpu.VMEM),
            pl.BlockSpec(memory_space=pltpu.SMEM),
        ],
        scratch_shapes=[
            pltpu.VMEM((N_DEV - 1, MQ, NH), jnp.bfloat16),
            pltpu.VMEM((N_DEV - 1, MQ, NH), jnp.bfloat16),
            pltpu.VMEM((MQ, NH), jnp.bfloat16),
            pltpu.VMEM((MQ, NH), jnp.bfloat16),
            pltpu.VMEM((MQ, NH), jnp.float32),
            pltpu.VMEM((MQ, NH), jnp.float32),
            pltpu.VMEM((N_DEV, 8, 128), jnp.float32),
            pltpu.SemaphoreType.DMA((N_DEV - 1,)),
            pltpu.SemaphoreType.DMA((N_DEV - 1,)),
            pltpu.SemaphoreType.DMA((N_DEV - 1,)),
            pltpu.SemaphoreType.DMA((N_DEV - 1,)),
            pltpu.SemaphoreType.DMA((N_DEV - 1,)),
            pltpu.SemaphoreType.DMA((N_DEV - 1,)),
            pltpu.SemaphoreType.DMA((N_DEV - 1,)),
            pltpu.SemaphoreType.DMA((N_DEV - 1,)),
            pltpu.SemaphoreType.DMA((N_DEV - 1,)),
            pltpu.SemaphoreType.DMA((N_DEV - 1,)),
        ],
        compiler_params=pltpu.CompilerParams(collective_id=0),
    )(xl, w16)

    qidx = jnp.arange(N_DEV)
    left_half = ycw[(my + 1 - qidx) % N_DEV].reshape(M, NH)
    right_half = yccw[(qidx - my + 1) % N_DEV].reshape(M, NH)
    out = jnp.concatenate([left_half, right_half], axis=1)
    return out.astype(jnp.float32) * scale[0, 0]


# device time: 163984 ns/iter; 1.6461x vs baseline; 1.0215x over previous
import jax
import jax.numpy as jnp
from jax import lax
from jax.experimental import pallas as pl
from jax.experimental.pallas import tpu as pltpu

N_DEV = 4
M = 4096
N = 2048
NH = N // 2
KS = 1024
MQ = M // N_DEV
RC = 4
RCM = MQ // RC


def _body(xl_ref, w_ref, ycw_ref, yccw_ref, scale_ref,
          rcv_cw, rcv_ccw, sb_cw, sb_ccw, own_cw, own_ccw, amax_ref,
          rs_s_cw, rs_r_cw, rs_s_ccw, rs_r_ccw,
          ax_s, ax_r, ag_s_cw, ag_r_cw, ag_s_ccw, ag_r_ccw):
    my = lax.axis_index("i")
    right = lax.rem(my + 1, N_DEV)
    left = lax.rem(my + N_DEV - 1, N_DEV)

    barrier = pltpu.get_barrier_semaphore()
    for nbr in (left, right):
        pl.semaphore_signal(barrier, inc=1, device_id=(nbr,),
                            device_id_type=pl.DeviceIdType.MESH)
    pl.semaphore_wait(barrier, 2)

    def qdot(block, colhalf, r):
        a, b = r * RCM, (r + 1) * RCM
        return jnp.dot(xl_ref[block, a:b, :],
                       w_ref[:, colhalf * NH:(colhalf + 1) * NH],
                       preferred_element_type=jnp.float32)

    for r in range(RC):
        a, b = r * RCM, (r + 1) * RCM
        sb_cw[a:b, :] = qdot(0, 0, r).astype(jnp.bfloat16)
        sb_ccw[a:b, :] = qdot(0, 1, r).astype(jnp.bfloat16)

    am_parts = []
    for s in range(N_DEV - 1):
        cw = pltpu.make_async_remote_copy(
            src_ref=sb_cw if s == 0 else rcv_cw.at[s - 1],
            dst_ref=rcv_cw.at[s],
            send_sem=rs_s_cw.at[s], recv_sem=rs_r_cw.at[s],
            device_id=(right,), device_id_type=pl.DeviceIdType.MESH)
        ccw = pltpu.make_async_remote_copy(
            src_ref=sb_ccw if s == 0 else rcv_ccw.at[s - 1],
            dst_ref=rcv_ccw.at[s],
            send_sem=rs_s_ccw.at[s], recv_sem=rs_r_ccw.at[s],
            device_id=(left,), device_id_type=pl.DeviceIdType.MESH)
        cw.start()
        ccw.start()
        for r in range(RC):
            a, b = r * RCM, (r + 1) * RCM
            own_cw[a:b, :] = qdot(s + 1, 0, r)
            own_ccw[a:b, :] = qdot(3 - s, 1, r)
        cw.wait()
        ccw.wait()
        for r in range(RC):
            a, b = r * RCM, (r + 1) * RCM
            tcw = rcv_cw[s, a:b, :].astype(jnp.float32) + own_cw[a:b, :]
            tccw = rcv_ccw[s, a:b, :].astype(jnp.float32) + own_ccw[a:b, :]
            if s < N_DEV - 2:
                rcv_cw[s, a:b, :] = tcw.astype(jnp.bfloat16)
                rcv_ccw[s, a:b, :] = tccw.astype(jnp.bfloat16)
            else:
                own_cw[a:b, :] = tcw
                own_ccw[a:b, :] = tccw
                am_parts.append(jnp.maximum(jnp.max(tcw), jnp.max(tccw)))

    am = am_parts[0]
    for p in am_parts[1:]:
        am = jnp.maximum(am, p)
    am = jnp.maximum(am, 0.0)
    amax_ref[0, :, :] = jnp.full((8, 128), am, dtype=jnp.float32)
    opp = lax.rem(my + 2, N_DEV)
    ax_rdmas = []
    for j, (tgt, slot) in enumerate(((right, 3), (left, 1), (opp, 2))):
        rdma = pltpu.make_async_remote_copy(
            src_ref=amax_ref.at[0], dst_ref=amax_ref.at[slot],
            send_sem=ax_s.at[j], recv_sem=ax_r.at[slot - 1],
            device_id=(tgt,), device_id_type=pl.DeviceIdType.MESH)
        rdma.start()
        ax_rdmas.append(rdma)
    for rdma in ax_rdmas:
        rdma.wait()
    g_amax = jnp.max(amax_ref[:, :, :])
    scale = g_amax / 127.0
    inv_scale = 127.0 / g_amax
    scale_ref[0, 0] = scale

    for r in range(RC):
        a, b = r * RCM, (r + 1) * RCM
        qcw = jnp.clip(jnp.round(jnp.maximum(own_cw[a:b, :], 0.0) * inv_scale),
                       0.0, 127.0)
        qccw = jnp.clip(jnp.round(jnp.maximum(own_ccw[a:b, :], 0.0) * inv_scale),
                        0.0, 127.0)
        ycw_ref[0, a:b, :] = qcw.astype(jnp.int8)
        yccw_ref[0, a:b, :] = qccw.astype(jnp.int8)
    for t in range(N_DEV - 1):
        cw = pltpu.make_async_remote_copy(
            src_ref=ycw_ref.at[t], dst_ref=ycw_ref.at[t + 1],
            send_sem=ag_s_cw.at[t], recv_sem=ag_r_cw.at[t],
            device_id=(right,), device_id_type=pl.DeviceIdType.MESH)
        ccw = pltpu.make_async_remote_copy(
            src_ref=yccw_ref.at[t], dst_ref=yccw_ref.at[t + 1],
            send_sem=ag_s_ccw.at[t], recv_sem=ag_r_ccw.at[t],
            device_id=(left,), device_id_type=pl.DeviceIdType.MESH)
        cw.start()
        ccw.start()
        cw.wait()
        ccw.wait()


def kernel(x, w_mat):
    x16 = x.astype(jnp.bfloat16)
    w16 = w_mat.astype(jnp.bfloat16)
    my = lax.axis_index("i")

    sidx = (my - jnp.arange(N_DEV)) % N_DEV
    xl = x16.reshape(N_DEV, MQ, KS)[sidx]

    ycw, yccw, scale = pl.pallas_call(
        _body,
        out_shape=[
            jax.ShapeDtypeStruct((N_DEV, MQ, NH), jnp.int8),
            jax.ShapeDtypeStruct((N_DEV, MQ, NH), jnp.int8),
            jax.ShapeDtypeStruct((1, 1), jnp.float32),
        ],
        in_specs=[
            pl.BlockSpec(memory_space=pltpu.VMEM),
            pl.BlockSpec(memory_space=pltpu.VMEM),
        ],
        out_specs=[
            pl.BlockSpec(memory_space=pltpu.VMEM),
            pl.BlockSpec(memory_space=pltpu.VMEM),
            pl.BlockSpec(memory_space=pltpu.SMEM),
        ],
        scratch_shapes=[
            pltpu.VMEM((N_DEV - 1, MQ, NH), jnp.bfloat16),
            pltpu.VMEM((N_DEV - 1, MQ, NH), jnp.bfloat16),
            pltpu.VMEM((MQ, NH), jnp.bfloat16),
            pltpu.VMEM((MQ, NH), jnp.bfloat16),
            pltpu.VMEM((MQ, NH), jnp.float32),
            pltpu.VMEM((MQ, NH), jnp.float32),
            pltpu.VMEM((N_DEV, 8, 128), jnp.float32),
            pltpu.SemaphoreType.DMA((N_DEV - 1,)),
            pltpu.SemaphoreType.DMA((N_DEV - 1,)),
            pltpu.SemaphoreType.DMA((N_DEV - 1,)),
            pltpu.SemaphoreType.DMA((N_DEV - 1,)),
            pltpu.SemaphoreType.DMA((N_DEV - 1,)),
            pltpu.SemaphoreType.DMA((N_DEV - 1,)),
            pltpu.SemaphoreType.DMA((N_DEV - 1,)),
            pltpu.SemaphoreType.DMA((N_DEV - 1,)),
            pltpu.SemaphoreType.DMA((N_DEV - 1,)),
            pltpu.SemaphoreType.DMA((N_DEV - 1,)),
        ],
        compiler_params=pltpu.CompilerParams(collective_id=0),
    )(xl, w16)

    qidx = jnp.arange(N_DEV)
    left_half = ycw[(my + 1 - qidx) % N_DEV].reshape(M, NH)
    right_half = yccw[(qidx - my + 1) % N_DEV].reshape(M, NH)
    out = jnp.concatenate([left_half, right_half], axis=1)
    return out.astype(jnp.float32) * scale[0, 0]


# device time: 156436 ns/iter; 1.7255x vs baseline; 1.0482x over previous
import jax
import jax.numpy as jnp
from jax import lax
from jax.experimental import pallas as pl
from jax.experimental.pallas import tpu as pltpu

N_DEV = 4
M = 4096
N = 2048
NH = N // 2
CC = 2
CH = NH // CC
KS = 1024
MQ = M // N_DEV
CW, CCW = 0, 1


def _body(xl_ref, w_ref, ycw_ref, yccw_ref, scale_ref,
          rcv_cw, rcv_ccw, sb_cw, sb_ccw, own_cw, own_ccw, amax_ref,
          rs_s_cw, rs_r_cw, rs_s_ccw, rs_r_ccw,
          ax_s, ax_r, ag_s_cw, ag_r_cw, ag_s_ccw, ag_r_ccw):
    my = lax.axis_index("i")
    right = lax.rem(my + 1, N_DEV)
    left = lax.rem(my + N_DEV - 1, N_DEV)

    barrier = pltpu.get_barrier_semaphore()
    for nbr in (left, right):
        pl.semaphore_signal(barrier, inc=1, device_id=(nbr,),
                            device_id_type=pl.DeviceIdType.MESH)
    pl.semaphore_wait(barrier, 2)

    def wslice(ring, c):
        base = ring * NH + c * CH
        return w_ref[:, base:base + CH]

    def rs_rdma(s, ring, c):
        rcv = rcv_cw if ring == CW else rcv_ccw
        sb = sb_cw if ring == CW else sb_ccw
        return pltpu.make_async_remote_copy(
            src_ref=sb.at[c] if s == 0 else rcv.at[s - 1, c],
            dst_ref=rcv.at[s, c],
            send_sem=(rs_s_cw if ring == CW else rs_s_ccw).at[s, c],
            recv_sem=(rs_r_cw if ring == CW else rs_r_ccw).at[s, c],
            device_id=(right,) if ring == CW else (left,),
            device_id_type=pl.DeviceIdType.MESH,
        )

    cur = {}
    for c in range(CC):
        sb_cw[c] = jnp.dot(xl_ref[0], wslice(CW, c),
                           preferred_element_type=jnp.float32
                           ).astype(jnp.bfloat16)
        sb_ccw[c] = jnp.dot(xl_ref[0], wslice(CCW, c),
                            preferred_element_type=jnp.float32
                            ).astype(jnp.bfloat16)
        for ring in (CW, CCW):
            d = rs_rdma(0, ring, c)
            d.start()
            cur[(ring, c)] = d

    am_parts = []
    for s in range(N_DEV - 1):
        nxt = {}
        for c in range(CC):
            for ring in (CW, CCW):
                blk = s + 1 if ring == CW else 3 - s
                part = jnp.dot(xl_ref[blk], wslice(ring, c),
                               preferred_element_type=jnp.float32)
                cur[(ring, c)].wait()
                rcv = rcv_cw if ring == CW else rcv_ccw
                tot = rcv[s, c].astype(jnp.float32) + part
                if s < N_DEV - 2:
                    rcv[s, c] = tot.astype(jnp.bfloat16)
                    d = rs_rdma(s + 1, ring, c)
                    d.start()
                    nxt[(ring, c)] = d
                else:
                    own = own_cw if ring == CW else own_ccw
                    own[c] = tot
                    am_parts.append(jnp.max(tot))
        cur = nxt

    am = am_parts[0]
    for p in am_parts[1:]:
        am = jnp.maximum(am, p)
    am = jnp.maximum(am, 0.0)
    amax_ref[0, :, :] = jnp.full((8, 128), am, dtype=jnp.float32)
    opp = lax.rem(my + 2, N_DEV)
    ax_rdmas = []
    for j, (tgt, slot) in enumerate(((right, 3), (left, 1), (opp, 2))):
        rdma = pltpu.make_async_remote_copy(
            src_ref=amax_ref.at[0], dst_ref=amax_ref.at[slot],
            send_sem=ax_s.at[j], recv_sem=ax_r.at[slot - 1],
            device_id=(tgt,), device_id_type=pl.DeviceIdType.MESH)
        rdma.start()
        ax_rdmas.append(rdma)
    for rdma in ax_rdmas:
        rdma.wait()
    g_amax = jnp.max(amax_ref[:, :, :])
    scale = g_amax / 127.0
    inv_scale = 127.0 / g_amax
    scale_ref[0, 0] = scale

    for c in range(CC):
        qcw = jnp.clip(jnp.round(jnp.maximum(own_cw[c], 0.0) * inv_scale),
                       0.0, 127.0)
        qccw = jnp.clip(jnp.round(jnp.maximum(own_ccw[c], 0.0) * inv_scale),
                        0.0, 127.0)
        ycw_ref[0, :, c * CH:(c + 1) * CH] = qcw.astype(jnp.int8)
        yccw_ref[0, :, c * CH:(c + 1) * CH] = qccw.astype(jnp.int8)
    for t in range(N_DEV - 1):
        cw = pltpu.make_async_remote_copy(
            src_ref=ycw_ref.at[t], dst_ref=ycw_ref.at[t + 1],
            send_sem=ag_s_cw.at[t], recv_sem=ag_r_cw.at[t],
            device_id=(right,), device_id_type=pl.DeviceIdType.MESH)
        ccw = pltpu.make_async_remote_copy(
            src_ref=yccw_ref.at[t], dst_ref=yccw_ref.at[t + 1],
            send_sem=ag_s_ccw.at[t], recv_sem=ag_r_ccw.at[t],
            device_id=(left,), device_id_type=pl.DeviceIdType.MESH)
        cw.start()
        ccw.start()
        cw.wait()
        ccw.wait()


def kernel(x, w_mat):
    x16 = x.astype(jnp.bfloat16)
    w16 = w_mat.astype(jnp.bfloat16)
    my = lax.axis_index("i")

    sidx = (my - jnp.arange(N_DEV)) % N_DEV
    xl = x16.reshape(N_DEV, MQ, KS)[sidx]

    ycw, yccw, scale = pl.pallas_call(
        _body,
        out_shape=[
            jax.ShapeDtypeStruct((N_DEV, MQ, NH), jnp.int8),
            jax.ShapeDtypeStruct((N_DEV, MQ, NH), jnp.int8),
            jax.ShapeDtypeStruct((1, 1), jnp.float32),
        ],
        in_specs=[
            pl.BlockSpec(memory_space=pltpu.VMEM),
            pl.BlockSpec(memory_space=pltpu.VMEM),
        ],
        out_specs=[
            pl.BlockSpec(memory_space=pltpu.VMEM),
            pl.BlockSpec(memory_space=pltpu.VMEM),
            pl.BlockSpec(memory_space=pltpu.SMEM),
        ],
        scratch_shapes=[
            pltpu.VMEM((N_DEV - 1, CC, MQ, CH), jnp.bfloat16),
            pltpu.VMEM((N_DEV - 1, CC, MQ, CH), jnp.bfloat16),
            pltpu.VMEM((CC, MQ, CH), jnp.bfloat16),
            pltpu.VMEM((CC, MQ, CH), jnp.bfloat16),
            pltpu.VMEM((CC, MQ, CH), jnp.float32),
            pltpu.VMEM((CC, MQ, CH), jnp.float32),
            pltpu.VMEM((N_DEV, 8, 128), jnp.float32),
            pltpu.SemaphoreType.DMA((N_DEV - 1, CC)),
            pltpu.SemaphoreType.DMA((N_DEV - 1, CC)),
            pltpu.SemaphoreType.DMA((N_DEV - 1, CC)),
            pltpu.SemaphoreType.DMA((N_DEV - 1, CC)),
            pltpu.SemaphoreType.DMA((N_DEV - 1,)),
            pltpu.SemaphoreType.DMA((N_DEV - 1,)),
            pltpu.SemaphoreType.DMA((N_DEV - 1,)),
            pltpu.SemaphoreType.DMA((N_DEV - 1,)),
            pltpu.SemaphoreType.DMA((N_DEV - 1,)),
            pltpu.SemaphoreType.DMA((N_DEV - 1,)),
        ],
        compiler_params=pltpu.CompilerParams(collective_id=0),
    )(xl, w16)

    qidx = jnp.arange(N_DEV)
    left_half = ycw[(my + 1 - qidx) % N_DEV].reshape(M, NH)
    right_half = yccw[(qidx - my + 1) % N_DEV].reshape(M, NH)
    out = jnp.concatenate([left_half, right_half], axis=1)
    return out.astype(jnp.float32) * scale[0, 0]
